# baseline (speedup 1.0000x reference)
import functools

import jax
import jax.numpy as jnp
from jax import lax
from jax.experimental import pallas as pl
from jax.experimental.pallas import tpu as pltpu
from jax.experimental.pallas import tpu_sc as plsc


def _gather_sc(idx, table, cb):
    n, = idx.shape
    v, d = table.shape
    assert d == 32
    info = plsc.get_sparse_core_info()
    nc = info.num_cores
    nw = nc * info.num_subcores
    n_per_w = n // nw
    blocks_per_w = n_per_w // 128
    a = cb * 128
    n_chunks = n_per_w // a
    ts = cb * 1024
    gs = (n // 128) * 1024
    mesh = plsc.VectorSubcoreMesh(core_axis_name="c", subcore_axis_name="s")

    @functools.partial(
        pl.kernel,
        mesh=mesh,
        out_type=jax.ShapeDtypeStruct((n * d,), jnp.float32),
        scratch_types=[
            pltpu.VMEM((n_per_w,), jnp.int32),
            pltpu.VMEM((a, d), jnp.float32),
            pltpu.VMEM((4 * ts,), jnp.float32),
            pltpu.SemaphoreType.DMA,
            pltpu.SemaphoreType.DMA,
            pltpu.SemaphoreType.DMA,
        ],
        compiler_params=pltpu.CompilerParams(
            use_tc_tiling_on_sc=False, needs_layout_passes=False),
    )
    def k(idx_hbm, table_hbm, out_hbm, idx_all, rows_v, rows_t, si, sg, so):
        wid = lax.axis_index("s") * nc + lax.axis_index("c")
        base = wid * n_per_w
        base_blk = wid * blocks_per_w
        pltpu.async_copy(
            idx_hbm.at[pl.ds(base, n_per_w)], idx_all, si).wait()

        jj = jnp.arange(16, dtype=jnp.int32)

        def out_slices(j):
            for g in range(4):
                yield (rows_t.at[pl.ds(g * ts, ts)],
                       out_hbm.at[pl.ds(
                           g * gs + (base_blk + j * cb) * 1024, ts)])

        @pl.loop(0, n_chunks)
        def _(j):
            pltpu.async_copy(
                table_hbm.at[idx_all.at[pl.ds(j * a, a)]], rows_v, sg).wait()

            @pl.when(j > 0)
            def _():
                for src, dst in out_slices(0):
                    pltpu.make_async_copy(src, dst, so).wait()

            for t in range(16):
                jrot = (t + jj) & 15
                d_t = (jrot >> 3) * ts + (jrot & 7) * 128 + jj
                col1 = jrot + 16

                @pl.loop(0, cb * 8, unroll=4)
                def _(m):
                    c = m >> 3
                    k16 = m & 7
                    rowv = jnp.full((16,), m * 16, jnp.int32) + jj
                    ev = jnp.full((16,), c * 1024 + k16 * 16, jnp.int32) + d_t
                    v0 = plsc.load_gather(rows_v, [rowv, jrot])
                    plsc.store_scatter(rows_t, [ev], v0)
                    v1 = plsc.load_gather(rows_v, [rowv, col1])
                    plsc.store_scatter(rows_t, [ev + (2 * ts)], v1)

            for src, dst in out_slices(j):
                pltpu.async_copy(src, dst, so)

        for src, dst in out_slices(0):
            pltpu.make_async_copy(src, dst, so).wait()

    return k(idx, table)


def kernel(inputs, table):
    bsz, h = inputs.shape
    _, d = table.shape
    n = bsz * h
    idx = inputs.reshape(n).astype(jnp.int32)
    out1d = _gather_sc(idx, table, cb=10)
    out4d = out1d.reshape(d // 8, n // 128, 8, 128)
    return out4d.transpose(1, 3, 0, 2).reshape(n, d)
